# baseline (speedup 1.0000x reference)
import functools
import math

import jax
import jax.numpy as jnp
from jax import lax
from jax.experimental import pallas as pl
from jax.experimental.pallas import tpu as pltpu
from jax.experimental.pallas import tpu_sc as plsc

EMB_DIM = 64
D_MODEL = 768
SCALE = math.sqrt(EMB_DIM)

NC = 2
NS = 16
NW = NC * NS

CHUNK = 256
L = 16


def _sc_gather(table2, idx2d, b_per_w):
    n_real = b_per_w // CHUNK
    n_iter = (n_real + 1) // 2
    n_pad = 2 * n_iter + 2
    mesh = plsc.VectorSubcoreMesh(core_axis_name="c", subcore_axis_name="s")

    @functools.partial(
        pl.kernel,
        mesh=mesh,
        out_type=jax.ShapeDtypeStruct((NW * b_per_w, 2 * EMB_DIM), jnp.float32),
        scratch_types=[
            pltpu.VMEM((b_per_w,), jnp.int32),
            pltpu.VMEM((n_pad * CHUNK,), jnp.int32),
            pltpu.VMEM((CHUNK, 2 * EMB_DIM), jnp.float32),
            pltpu.VMEM((CHUNK, 2 * EMB_DIM), jnp.float32),
            pltpu.SemaphoreType.DMA,
            pltpu.SemaphoreType.DMA,
        ],
        compiler_params=pltpu.CompilerParams(needs_layout_passes=False),
    )
    def gather_kernel(idx_hbm, table_hbm, out_hbm, idx_v, q_v, buf0, buf1,
                      sem0, sem1):
        wid = lax.axis_index("s") * NC + lax.axis_index("c")
        base = wid * b_per_w
        pltpu.sync_copy(idx_hbm.at[wid], idx_v)

        zero16i = jnp.zeros((L,), jnp.int32)
        zero16f = jnp.zeros((L,), jnp.float32)
        iota16 = lax.iota(jnp.int32, L)

        def qbody(i, carry):
            v = idx_v[pl.ds(i * L, L)]
            q_v[pl.ds(i * L, L)] = jax.lax.shift_right_logical(v, 1)
            return carry

        lax.fori_loop(0, b_per_w // L, qbody, 0)

        def qpad(i, carry):
            q_v[pl.ds(b_per_w + i * L, L)] = zero16i
            return carry

        lax.fori_loop(0, (n_pad * CHUNK - b_per_w) // L, qpad, 0)

        def start_gather(c, buf, sem):
            return pltpu.async_copy(
                table_hbm.at[q_v.at[pl.ds(c * CHUNK, CHUNK)]], buf, sem
            )

        def zero_half(c, buf):
            def zbody(g, carry2):
                vi = idx_v[pl.ds(c * CHUNK + g * L, L)]
                colbase = EMB_DIM - (vi & 1) * EMB_DIM
                rowid = g * L + iota16
                for m in range(EMB_DIM):
                    plsc.store_scatter(buf, [rowid, colbase + m], zero16f)
                return carry2

            lax.fori_loop(0, CHUNK // L, zbody, 0)

        def finish(c, buf, sem):
            pltpu.make_async_copy(
                table_hbm.at[q_v.at[pl.ds(c * CHUNK, CHUNK)]], buf, sem
            ).wait()

            @pl.when(c < n_real)
            def _():
                zero_half(c, buf)
                pltpu.sync_copy(buf, out_hbm.at[pl.ds(base + c * CHUNK, CHUNK)])

        start_gather(0, buf0, sem0)

        def body(i, carry):
            c0 = 2 * i
            start_gather(c0 + 1, buf1, sem1)
            finish(c0, buf0, sem0)
            start_gather(c0 + 2, buf0, sem0)
            finish(c0 + 1, buf1, sem1)
            return carry

        lax.fori_loop(0, n_iter, body, 0)
        pltpu.make_async_copy(
            table_hbm.at[q_v.at[pl.ds(2 * n_iter * CHUNK, CHUNK)]], buf0, sem0
        ).wait()

    return gather_kernel(idx2d, table2)


def _mm_body(x_ref, w_ref, b_ref, o_ref):
    acc = jnp.dot(x_ref[...], w_ref[...], preferred_element_type=jnp.float32)
    o_ref[...] = acc + b_ref[...]


def _tc_project(gathered, w2, b2, block_m):
    n = gathered.shape[0]
    return pl.pallas_call(
        _mm_body,
        grid=(n // block_m,),
        in_specs=[
            pl.BlockSpec((block_m, 2 * EMB_DIM), lambda i: (i, 0)),
            pl.BlockSpec((2 * EMB_DIM, D_MODEL), lambda i: (0, 0)),
            pl.BlockSpec((1, D_MODEL), lambda i: (0, 0)),
        ],
        out_specs=pl.BlockSpec((block_m, D_MODEL), lambda i: (i, 0)),
        out_shape=jax.ShapeDtypeStruct((n, D_MODEL), jnp.float32),
    )(gathered, w2, b2)


def kernel(persona, emb_table, proj_w, proj_b):
    batch, hist = persona.shape
    n = batch * hist
    b_per_w = n // NW
    table2 = emb_table.reshape(emb_table.shape[0] // 2, 2 * EMB_DIM)
    idx2d = persona.astype(jnp.int32).T.reshape(NW, b_per_w)
    gathered = _sc_gather(table2, idx2d, b_per_w)
    wt8 = jnp.transpose(proj_w) * SCALE
    w2 = jnp.concatenate([wt8, wt8], axis=0)
    out2d = _tc_project(gathered, w2, proj_b.reshape(1, D_MODEL), 2048)
    return out2d.reshape(hist, batch, D_MODEL).transpose(1, 0, 2)


# speedup vs baseline: 1.3823x; 1.3823x over previous
import functools
import math

import jax
import jax.numpy as jnp
from jax import lax
from jax.experimental import pallas as pl
from jax.experimental.pallas import tpu as pltpu
from jax.experimental.pallas import tpu_sc as plsc

EMB_DIM = 64
D_MODEL = 768
SCALE = math.sqrt(EMB_DIM)

NC = 2
NS = 16
NW = NC * NS

CHUNK = 128
L = 16


def _sc_gather(table2, idx2d, b_per_w):
    n_real = b_per_w // CHUNK
    n_iter = (n_real + 1) // 2
    n_pad = 2 * n_iter + 2
    mesh = plsc.VectorSubcoreMesh(core_axis_name="c", subcore_axis_name="s")

    @functools.partial(
        pl.kernel,
        mesh=mesh,
        out_type=jax.ShapeDtypeStruct((NW * b_per_w, 2 * EMB_DIM), jnp.float32),
        scratch_types=[
            pltpu.VMEM((b_per_w,), jnp.int32),
            pltpu.VMEM((n_pad * CHUNK,), jnp.int32),
            pltpu.VMEM((CHUNK, 2 * EMB_DIM), jnp.float32),
            pltpu.VMEM((CHUNK, 2 * EMB_DIM), jnp.float32),
            pltpu.SemaphoreType.DMA,
            pltpu.SemaphoreType.DMA,
        ],
        compiler_params=pltpu.CompilerParams(needs_layout_passes=False),
    )
    def gather_kernel(idx_hbm, table_hbm, out_hbm, idx_v, q_v, buf0, buf1,
                      sem0, sem1):
        wid = lax.axis_index("s") * NC + lax.axis_index("c")
        base = wid * b_per_w
        pltpu.sync_copy(idx_hbm.at[wid], idx_v)

        zero16i = jnp.zeros((L,), jnp.int32)
        zero16f = jnp.zeros((L,), jnp.float32)
        iota16 = lax.iota(jnp.int32, L)

        def qbody(i, carry):
            v = idx_v[pl.ds(i * L, L)]
            q_v[pl.ds(i * L, L)] = jax.lax.shift_right_logical(v, 1)
            return carry

        lax.fori_loop(0, b_per_w // L, qbody, 0)

        def qpad(i, carry):
            q_v[pl.ds(b_per_w + i * L, L)] = zero16i
            return carry

        lax.fori_loop(0, (n_pad * CHUNK - b_per_w) // L, qpad, 0)

        def start_gather(c, buf, sem):
            return pltpu.async_copy(
                table_hbm.at[q_v.at[pl.ds(c * CHUNK, CHUNK)]], buf, sem
            )

        def zero_half(c, buf):
            def zbody(g, carry2):
                vi = idx_v[pl.ds(c * CHUNK + g * L, L)]
                colbase = EMB_DIM - (vi & 1) * EMB_DIM
                rowid = g * L + iota16
                for m in range(EMB_DIM):
                    plsc.store_scatter(buf, [rowid, colbase + m], zero16f)
                return carry2

            lax.fori_loop(0, CHUNK // L, zbody, 0)

        def finish(c, buf, sem):
            pltpu.make_async_copy(
                table_hbm.at[q_v.at[pl.ds(c * CHUNK, CHUNK)]], buf, sem
            ).wait()

            @pl.when(c < n_real)
            def _():
                zero_half(c, buf)
                pltpu.sync_copy(buf, out_hbm.at[pl.ds(base + c * CHUNK, CHUNK)])

        start_gather(0, buf0, sem0)

        def body(i, carry):
            c0 = 2 * i
            start_gather(c0 + 1, buf1, sem1)
            finish(c0, buf0, sem0)
            start_gather(c0 + 2, buf0, sem0)
            finish(c0 + 1, buf1, sem1)
            return carry

        lax.fori_loop(0, n_iter, body, 0)
        pltpu.make_async_copy(
            table_hbm.at[q_v.at[pl.ds(2 * n_iter * CHUNK, CHUNK)]], buf0, sem0
        ).wait()

    return gather_kernel(idx2d, table2)


def _mm_body(x_ref, w_ref, b_ref, o_ref):
    acc = jnp.dot(x_ref[...], w_ref[...], preferred_element_type=jnp.float32)
    o_ref[...] = acc + b_ref[...]


def _tc_project(gathered, w2, b2, block_m):
    n = gathered.shape[0]
    return pl.pallas_call(
        _mm_body,
        grid=(n // block_m,),
        in_specs=[
            pl.BlockSpec((block_m, 2 * EMB_DIM), lambda i: (i, 0)),
            pl.BlockSpec((2 * EMB_DIM, D_MODEL), lambda i: (0, 0)),
            pl.BlockSpec((1, D_MODEL), lambda i: (0, 0)),
        ],
        out_specs=pl.BlockSpec((block_m, D_MODEL), lambda i: (i, 0)),
        out_shape=jax.ShapeDtypeStruct((n, D_MODEL), jnp.float32),
    )(gathered, w2, b2)


def kernel(persona, emb_table, proj_w, proj_b):
    batch, hist = persona.shape
    n = batch * hist
    b_per_w = n // NW
    table2 = emb_table.reshape(emb_table.shape[0] // 2, 2 * EMB_DIM)
    idx2d = persona.astype(jnp.int32).T.reshape(NW, b_per_w)
    gathered = _sc_gather(table2, idx2d, b_per_w)
    wt8 = jnp.transpose(proj_w) * SCALE
    w2 = jnp.concatenate([wt8, wt8], axis=0)
    out2d = _tc_project(gathered, w2, proj_b.reshape(1, D_MODEL), 2048)
    return out2d.reshape(hist, batch, D_MODEL).transpose(1, 0, 2)
